# baseline (speedup 1.0000x reference)
import functools

import jax
import jax.numpy as jnp
from jax import lax
from jax.experimental import pallas as pl
from jax.experimental.pallas import tpu as pltpu
from jax.experimental.pallas import tpu_sc as plsc

NC = 2
NS = 16
NWORK = NC * NS
L = 16
RW = 128


def _sc_scores(pin, pout, i_idx, o_idx, n_idx, B, E, W, N):
    PK = RW // E
    b_per = B // NWORK
    C = 64
    n_ch = b_per // C
    G = C // L

    mesh = plsc.VectorSubcoreMesh(core_axis_name="c", subcore_axis_name="s")

    @functools.partial(
        pl.kernel,
        out_type=(
            jax.ShapeDtypeStruct((B,), jnp.float32),
            jax.ShapeDtypeStruct((B,), jnp.float32),
        ),
        mesh=mesh,
        compiler_params=pltpu.CompilerParams(needs_layout_passes=False),
        scratch_types=[
            pltpu.VMEM((C,), jnp.int32),
            pltpu.VMEM((C * W,), jnp.int32),
            pltpu.VMEM((C * N,), jnp.int32),
            pltpu.VMEM((C,), jnp.int32),
            pltpu.VMEM((C * W,), jnp.int32),
            pltpu.VMEM((C * N,), jnp.int32),
            pltpu.VMEM((C, RW), jnp.float32),
            pltpu.VMEM((C * W, RW), jnp.float32),
            pltpu.VMEM((C * N, RW), jnp.float32),
            pltpu.VMEM((C,), jnp.float32),
            pltpu.VMEM((C,), jnp.float32),
            pltpu.SemaphoreType.DMA,
        ],
    )
    def k(pin_hbm, pout_hbm, ii_hbm, oi_hbm, ni_hbm, spos_hbm, sneg_hbm,
          iv, ov, nv, ivs, ovs, nvs, irows, orows, nrows, sp, sn, sem):
        wid = lax.axis_index("s") * NC + lax.axis_index("c")

        def chunk(c, _):
            base = wid * b_per + c * C
            pltpu.sync_copy(ii_hbm.at[pl.ds(base, C)], iv)
            pltpu.sync_copy(oi_hbm.at[pl.ds(base * W, C * W)], ov)
            pltpu.sync_copy(ni_hbm.at[pl.ds(base * N, C * N)], nv)
            for kk in range(C // L):
                s = pl.ds(kk * L, L)
                ivs[s] = lax.shift_right_logical(iv[s], 2)
            for kk in range(C * W // L):
                s = pl.ds(kk * L, L)
                ovs[s] = lax.shift_right_logical(ov[s], 2)
            for kk in range(C * N // L):
                s = pl.ds(kk * L, L)
                nvs[s] = lax.shift_right_logical(nv[s], 2)
            cp_i = pltpu.async_copy(pin_hbm.at[ivs], irows, sem)
            cp_o = pltpu.async_copy(pout_hbm.at[ovs], orows, sem)
            cp_n = pltpu.async_copy(pout_hbm.at[nvs], nrows, sem)
            cp_i.wait()
            cp_o.wait()
            cp_n.wait()

            def group(g, _):
                bvec = g * L + lax.iota(jnp.int32, L)
                icol = (plsc.load_gather(iv, [bvec]) & (PK - 1)) * E
                ocols = []
                for w in range(W):
                    ocols.append(
                        (plsc.load_gather(ov, [bvec * W + w]) & (PK - 1)) * E)
                ncols = []
                for n in range(N):
                    ncols.append(
                        (plsc.load_gather(nv, [bvec * N + n]) & (PK - 1)) * E)
                pacc = jnp.zeros((L,), jnp.float32)
                nacc = jnp.zeros((L,), jnp.float32)
                for e in range(E):
                    v_in = plsc.load_gather(irows, [bvec, icol + e])
                    pe = plsc.load_gather(orows, [bvec * W, ocols[0] + e])
                    for w in range(1, W):
                        pe = pe + plsc.load_gather(
                            orows, [bvec * W + w, ocols[w] + e])
                    ne = plsc.load_gather(nrows, [bvec * N, ncols[0] + e])
                    for n in range(1, N):
                        ne = ne + plsc.load_gather(
                            nrows, [bvec * N + n, ncols[n] + e])
                    pacc = pacc + v_in * pe
                    nacc = nacc + v_in * ne
                sp[pl.ds(g * L, L)] = pacc
                sn[pl.ds(g * L, L)] = nacc
                return 0

            lax.fori_loop(0, G, group, 0)
            pltpu.sync_copy(sp, spos_hbm.at[pl.ds(base, C)])
            pltpu.sync_copy(sn, sneg_hbm.at[pl.ds(base, C)])
            return 0

        lax.fori_loop(0, n_ch, chunk, 0)

    return k(pin, pout, i_idx, o_idx, n_idx)


def _tc_loss(s_pos, s_neg, B):
    def body(sp_ref, sn_ref, o_ref):
        def logsig(x):
            return jnp.minimum(x, 0.0) - jnp.log1p(jnp.exp(-jnp.abs(x)))

        o_ref[0, 0] = jnp.sum(logsig(sn_ref[...]) - logsig(sp_ref[...])) \
            * (1.0 / B)

    return pl.pallas_call(
        body,
        in_specs=[
            pl.BlockSpec(memory_space=pltpu.VMEM),
            pl.BlockSpec(memory_space=pltpu.VMEM),
        ],
        out_specs=pl.BlockSpec(memory_space=pltpu.SMEM),
        out_shape=jax.ShapeDtypeStruct((1, 1), jnp.float32),
    )(s_pos, s_neg)


def kernel(i, o, neg, in_table, out_table):
    B = i.shape[0]
    W = o.shape[1]
    N = neg.shape[1]
    E = in_table.shape[1]
    pin = in_table.reshape(-1, RW)
    pout = out_table.reshape(-1, RW)
    i32 = i.astype(jnp.int32)
    o32 = o.astype(jnp.int32).reshape(-1)
    n32 = neg.astype(jnp.int32).reshape(-1)
    s_pos, s_neg = _sc_scores(pin, pout, i32, o32, n32, B, E, W, N)
    loss = _tc_loss(s_pos.reshape(128, -1), s_neg.reshape(128, -1), B)
    return loss[0, 0]


# speedup vs baseline: 1.5032x; 1.5032x over previous
import functools

import jax
import jax.numpy as jnp
from jax import lax
from jax.experimental import pallas as pl
from jax.experimental.pallas import tpu as pltpu
from jax.experimental.pallas import tpu_sc as plsc

NC = 2
NS = 16
NWORK = NC * NS
L = 16
RW = 128
K4 = 2048
NB = 123
QP = K4 * NB


def _tc_repack(tt, E):
    def body(i0, i1, i2, i3, out_ref):
        ys = [i[...].T for i in (i0, i1, i2, i3)]
        out_ref[...] = jnp.concatenate(ys, axis=1)

    last_blk = 488

    return pl.pallas_call(
        body,
        grid=(NB,),
        in_specs=[
            pl.BlockSpec(
                (E, K4),
                lambda g, j=j: (0, jnp.minimum(j * NB + g, last_blk)))
            for j in range(4)
        ],
        out_specs=pl.BlockSpec((K4, RW), lambda g: (g, 0)),
        out_shape=jax.ShapeDtypeStruct((QP, RW), jnp.float32),
    )(tt, tt, tt, tt)


def _group_of(v):
    one = jnp.int32(1)
    zero = jnp.int32(0)
    return (jnp.where(v >= QP, one, zero)
            + jnp.where(v >= 2 * QP, one, zero)
            + jnp.where(v >= 3 * QP, one, zero))


def _sc_scores(pin, pout, i_idx, o_idx, n_idx, B, E, W, N):
    b_per = B // NWORK
    C = 64
    n_ch = b_per // C
    G = C // L

    mesh = plsc.VectorSubcoreMesh(core_axis_name="c", subcore_axis_name="s")

    @functools.partial(
        pl.kernel,
        out_type=(
            jax.ShapeDtypeStruct((B,), jnp.float32),
            jax.ShapeDtypeStruct((B,), jnp.float32),
        ),
        mesh=mesh,
        compiler_params=pltpu.CompilerParams(needs_layout_passes=False),
        scratch_types=[
            pltpu.VMEM((C,), jnp.int32),
            pltpu.VMEM((C * W,), jnp.int32),
            pltpu.VMEM((C * N,), jnp.int32),
            pltpu.VMEM((C,), jnp.int32),
            pltpu.VMEM((C * W,), jnp.int32),
            pltpu.VMEM((C * N,), jnp.int32),
            pltpu.VMEM((C, RW), jnp.float32),
            pltpu.VMEM((C * W, RW), jnp.float32),
            pltpu.VMEM((C * N, RW), jnp.float32),
            pltpu.VMEM((C,), jnp.float32),
            pltpu.VMEM((C,), jnp.float32),
            pltpu.SemaphoreType.DMA,
        ],
    )
    def k(pin_hbm, pout_hbm, ii_hbm, oi_hbm, ni_hbm, spos_hbm, sneg_hbm,
          iv, ov, nv, ivs, ovs, nvs, irows, orows, nrows, sp, sn, sem):
        wid = lax.axis_index("s") * NC + lax.axis_index("c")

        def chunk(c, _):
            base = wid * b_per + c * C
            pltpu.sync_copy(ii_hbm.at[pl.ds(base, C)], iv)
            pltpu.sync_copy(oi_hbm.at[pl.ds(base * W, C * W)], ov)
            pltpu.sync_copy(ni_hbm.at[pl.ds(base * N, C * N)], nv)
            for src, dst, nv_ in ((iv, ivs, C), (ov, ovs, C * W),
                                  (nv, nvs, C * N)):
                for kk in range(nv_ // L):
                    s = pl.ds(kk * L, L)
                    v = src[s]
                    dst[s] = v - _group_of(v) * QP
            cp_i = pltpu.async_copy(pin_hbm.at[ivs], irows, sem)
            cp_o = pltpu.async_copy(pout_hbm.at[ovs], orows, sem)
            cp_n = pltpu.async_copy(pout_hbm.at[nvs], nrows, sem)
            cp_i.wait()
            cp_o.wait()
            cp_n.wait()

            def group(g, _):
                bvec = g * L + lax.iota(jnp.int32, L)
                icol = _group_of(plsc.load_gather(iv, [bvec])) * E
                ocols = []
                for w in range(W):
                    ocols.append(
                        _group_of(plsc.load_gather(ov, [bvec * W + w])) * E)
                ncols = []
                for n in range(N):
                    ncols.append(
                        _group_of(plsc.load_gather(nv, [bvec * N + n])) * E)
                pacc = jnp.zeros((L,), jnp.float32)
                nacc = jnp.zeros((L,), jnp.float32)
                for e in range(E):
                    v_in = plsc.load_gather(irows, [bvec, icol + e])
                    pe = plsc.load_gather(orows, [bvec * W, ocols[0] + e])
                    for w in range(1, W):
                        pe = pe + plsc.load_gather(
                            orows, [bvec * W + w, ocols[w] + e])
                    ne = plsc.load_gather(nrows, [bvec * N, ncols[0] + e])
                    for n in range(1, N):
                        ne = ne + plsc.load_gather(
                            nrows, [bvec * N + n, ncols[n] + e])
                    pacc = pacc + v_in * pe
                    nacc = nacc + v_in * ne
                sp[pl.ds(g * L, L)] = pacc
                sn[pl.ds(g * L, L)] = nacc
                return 0

            lax.fori_loop(0, G, group, 0)
            pltpu.sync_copy(sp, spos_hbm.at[pl.ds(base, C)])
            pltpu.sync_copy(sn, sneg_hbm.at[pl.ds(base, C)])
            return 0

        lax.fori_loop(0, n_ch, chunk, 0)

    return k(pin, pout, i_idx, o_idx, n_idx)


def _tc_loss(s_pos, s_neg, B):
    def body(sp_ref, sn_ref, o_ref):
        def logsig(x):
            return jnp.minimum(x, 0.0) - jnp.log1p(jnp.exp(-jnp.abs(x)))

        o_ref[0, 0] = jnp.sum(logsig(sn_ref[...]) - logsig(sp_ref[...])) \
            * (1.0 / B)

    return pl.pallas_call(
        body,
        in_specs=[
            pl.BlockSpec(memory_space=pltpu.VMEM),
            pl.BlockSpec(memory_space=pltpu.VMEM),
        ],
        out_specs=pl.BlockSpec(memory_space=pltpu.SMEM),
        out_shape=jax.ShapeDtypeStruct((1, 1), jnp.float32),
    )(s_pos, s_neg)


def kernel(i, o, neg, in_table, out_table):
    B = i.shape[0]
    W = o.shape[1]
    N = neg.shape[1]
    E = in_table.shape[1]
    pin = _tc_repack(in_table.T, E)
    pout = _tc_repack(out_table.T, E)
    i32 = i.astype(jnp.int32)
    o32 = o.astype(jnp.int32).reshape(-1)
    n32 = neg.astype(jnp.int32).reshape(-1)
    s_pos, s_neg = _sc_scores(pin, pout, i32, o32, n32, B, E, W, N)
    loss = _tc_loss(s_pos.reshape(128, -1), s_neg.reshape(128, -1), B)
    return loss[0, 0]


# speedup vs baseline: 1.6723x; 1.1125x over previous
import functools

import jax
import jax.numpy as jnp
from jax import lax
from jax.experimental import pallas as pl
from jax.experimental.pallas import tpu as pltpu
from jax.experimental.pallas import tpu_sc as plsc

NC = 2
NS = 16
NWORK = NC * NS
L = 16
RW = 128
K4 = 2048
NB = 123
QP = K4 * NB


def _tc_repack(tt, E):
    def body(i0, i1, i2, i3, out_ref):
        eye = (lax.broadcasted_iota(jnp.int32, (E, E), 0)
               == lax.broadcasted_iota(jnp.int32, (E, E), 1)
               ).astype(jnp.float32)
        dn = (((0,), (0,)), ((), ()))
        ys = [lax.dot_general(i[...], eye, dn,
                              preferred_element_type=jnp.float32)
              for i in (i0, i1, i2, i3)]
        out_ref[...] = jnp.concatenate(ys, axis=1)

    last_blk = 488

    return pl.pallas_call(
        body,
        grid=(NB,),
        in_specs=[
            pl.BlockSpec(
                (E, K4),
                lambda g, j=j: (0, jnp.minimum(j * NB + g, last_blk)))
            for j in range(4)
        ],
        out_specs=pl.BlockSpec((K4, RW), lambda g: (g, 0)),
        out_shape=jax.ShapeDtypeStruct((QP, RW), jnp.float32),
    )(tt, tt, tt, tt)


def _group_of(v):
    one = jnp.int32(1)
    zero = jnp.int32(0)
    return (jnp.where(v >= QP, one, zero)
            + jnp.where(v >= 2 * QP, one, zero)
            + jnp.where(v >= 3 * QP, one, zero))


def _sc_scores(pin, pout, i_idx, o_idx, n_idx, B, E, W, N):
    b_per = B // NWORK
    C = 64
    n_ch = b_per // C
    G = C // L

    mesh = plsc.VectorSubcoreMesh(core_axis_name="c", subcore_axis_name="s")

    @functools.partial(
        pl.kernel,
        out_type=(
            jax.ShapeDtypeStruct((B,), jnp.float32),
            jax.ShapeDtypeStruct((B,), jnp.float32),
        ),
        mesh=mesh,
        compiler_params=pltpu.CompilerParams(needs_layout_passes=False),
        scratch_types=[
            pltpu.VMEM((C,), jnp.int32),
            pltpu.VMEM((C * W,), jnp.int32),
            pltpu.VMEM((C * N,), jnp.int32),
            pltpu.VMEM((C,), jnp.int32),
            pltpu.VMEM((C * W,), jnp.int32),
            pltpu.VMEM((C * N,), jnp.int32),
            pltpu.VMEM((C, RW), jnp.float32),
            pltpu.VMEM((C * W, RW), jnp.float32),
            pltpu.VMEM((C * N, RW), jnp.float32),
            pltpu.VMEM((C,), jnp.float32),
            pltpu.VMEM((C,), jnp.float32),
            pltpu.SemaphoreType.DMA,
        ],
    )
    def k(pin_hbm, pout_hbm, ii_hbm, oi_hbm, ni_hbm, spos_hbm, sneg_hbm,
          iv, ov, nv, ivs, ovs, nvs, irows, orows, nrows, sp, sn, sem):
        wid = lax.axis_index("s") * NC + lax.axis_index("c")

        def chunk(c, _):
            base = wid * b_per + c * C
            pltpu.sync_copy(ii_hbm.at[pl.ds(base, C)], iv)
            pltpu.sync_copy(oi_hbm.at[pl.ds(base * W, C * W)], ov)
            pltpu.sync_copy(ni_hbm.at[pl.ds(base * N, C * N)], nv)
            for src, dst, nv_ in ((iv, ivs, C), (ov, ovs, C * W),
                                  (nv, nvs, C * N)):
                for kk in range(nv_ // L):
                    s = pl.ds(kk * L, L)
                    v = src[s]
                    dst[s] = v - _group_of(v) * QP
            cp_i = pltpu.async_copy(pin_hbm.at[ivs], irows, sem)
            cp_o = pltpu.async_copy(pout_hbm.at[ovs], orows, sem)
            cp_n = pltpu.async_copy(pout_hbm.at[nvs], nrows, sem)
            cp_i.wait()
            cp_o.wait()
            cp_n.wait()

            def group(g, _):
                bvec = g * L + lax.iota(jnp.int32, L)
                icol = _group_of(plsc.load_gather(iv, [bvec])) * E
                ocols = []
                for w in range(W):
                    ocols.append(
                        _group_of(plsc.load_gather(ov, [bvec * W + w])) * E)
                ncols = []
                for n in range(N):
                    ncols.append(
                        _group_of(plsc.load_gather(nv, [bvec * N + n])) * E)
                pacc = jnp.zeros((L,), jnp.float32)
                nacc = jnp.zeros((L,), jnp.float32)
                lane = lax.iota(jnp.int32, L)
                for e in range(E):
                    ev = jnp.bitwise_and(e + lane, E - 1)
                    v_in = plsc.load_gather(irows, [bvec, icol + ev])
                    pe = plsc.load_gather(orows, [bvec * W, ocols[0] + ev])
                    for w in range(1, W):
                        pe = pe + plsc.load_gather(
                            orows, [bvec * W + w, ocols[w] + ev])
                    ne = plsc.load_gather(nrows, [bvec * N, ncols[0] + ev])
                    for n in range(1, N):
                        ne = ne + plsc.load_gather(
                            nrows, [bvec * N + n, ncols[n] + ev])
                    pacc = pacc + v_in * pe
                    nacc = nacc + v_in * ne
                sp[pl.ds(g * L, L)] = pacc
                sn[pl.ds(g * L, L)] = nacc
                return 0

            lax.fori_loop(0, G, group, 0)
            pltpu.sync_copy(sp, spos_hbm.at[pl.ds(base, C)])
            pltpu.sync_copy(sn, sneg_hbm.at[pl.ds(base, C)])
            return 0

        lax.fori_loop(0, n_ch, chunk, 0)

    return k(pin, pout, i_idx, o_idx, n_idx)


def _tc_loss(s_pos, s_neg, B):
    def body(sp_ref, sn_ref, o_ref):
        def logsig(x):
            return jnp.minimum(x, 0.0) - jnp.log1p(jnp.exp(-jnp.abs(x)))

        o_ref[0, 0] = jnp.sum(logsig(sn_ref[...]) - logsig(sp_ref[...])) \
            * (1.0 / B)

    return pl.pallas_call(
        body,
        in_specs=[
            pl.BlockSpec(memory_space=pltpu.VMEM),
            pl.BlockSpec(memory_space=pltpu.VMEM),
        ],
        out_specs=pl.BlockSpec(memory_space=pltpu.SMEM),
        out_shape=jax.ShapeDtypeStruct((1, 1), jnp.float32),
    )(s_pos, s_neg)


def kernel(i, o, neg, in_table, out_table):
    B = i.shape[0]
    W = o.shape[1]
    N = neg.shape[1]
    E = in_table.shape[1]
    pin = _tc_repack(in_table.T, E)
    pout = _tc_repack(out_table.T, E)
    i32 = i.astype(jnp.int32)
    o32 = o.astype(jnp.int32).reshape(-1)
    n32 = neg.astype(jnp.int32).reshape(-1)
    s_pos, s_neg = _sc_scores(pin, pout, i32, o32, n32, B, E, W, N)
    loss = _tc_loss(s_pos.reshape(128, -1), s_neg.reshape(128, -1), B)
    return loss[0, 0]


# speedup vs baseline: 2.2633x; 1.3534x over previous
import functools

import jax
import jax.numpy as jnp
from jax import lax
from jax.experimental import pallas as pl
from jax.experimental.pallas import tpu as pltpu
from jax.experimental.pallas import tpu_sc as plsc

NC = 2
NS = 16
NWORK = NC * NS
L = 16
RW = 128
K4 = 2048
NB = 123
QP = K4 * NB


def _tc_repack(tt, E):
    def body(i0, i1, i2, i3, out_ref):
        xcat = jnp.concatenate(
            [i0[...], i1[...], i2[...], i3[...]], axis=0)
        eye = (lax.broadcasted_iota(jnp.int32, (RW, RW), 0)
               == lax.broadcasted_iota(jnp.int32, (RW, RW), 1)
               ).astype(jnp.float32)
        dn = (((0,), (0,)), ((), ()))
        out_ref[...] = lax.dot_general(
            xcat, eye, dn, preferred_element_type=jnp.float32,
            precision=lax.Precision.HIGHEST)

    last_blk = 488

    return pl.pallas_call(
        body,
        grid=(NB,),
        in_specs=[
            pl.BlockSpec(
                (E, K4),
                lambda g, j=j: (0, jnp.minimum(j * NB + g, last_blk)))
            for j in range(4)
        ],
        out_specs=pl.BlockSpec((K4, RW), lambda g: (g, 0)),
        out_shape=jax.ShapeDtypeStruct((QP, RW), jnp.float32),
    )(tt, tt, tt, tt)


def _group_of(v):
    one = jnp.int32(1)
    zero = jnp.int32(0)
    return (jnp.where(v >= QP, one, zero)
            + jnp.where(v >= 2 * QP, one, zero)
            + jnp.where(v >= 3 * QP, one, zero))


def _sc_scores(pin, pout, i_idx, o_idx, n_idx, B, E, W, N):
    b_per = B // NWORK
    C = 64
    n_ch = b_per // C
    G = C // L

    mesh = plsc.VectorSubcoreMesh(core_axis_name="c", subcore_axis_name="s")

    @functools.partial(
        pl.kernel,
        out_type=(
            jax.ShapeDtypeStruct((B,), jnp.float32),
            jax.ShapeDtypeStruct((B,), jnp.float32),
        ),
        mesh=mesh,
        compiler_params=pltpu.CompilerParams(needs_layout_passes=False),
        scratch_types=[
            pltpu.VMEM((C,), jnp.int32),
            pltpu.VMEM((C * W,), jnp.int32),
            pltpu.VMEM((C * N,), jnp.int32),
            pltpu.VMEM((C,), jnp.int32),
            pltpu.VMEM((C * W,), jnp.int32),
            pltpu.VMEM((C * N,), jnp.int32),
            pltpu.VMEM((C, RW), jnp.float32),
            pltpu.VMEM((C * W, RW), jnp.float32),
            pltpu.VMEM((C * N, RW), jnp.float32),
            pltpu.VMEM((C,), jnp.float32),
            pltpu.VMEM((C,), jnp.float32),
            pltpu.SemaphoreType.DMA,
        ],
    )
    def k(pin_hbm, pout_hbm, ii_hbm, oi_hbm, ni_hbm, spos_hbm, sneg_hbm,
          iv, ov, nv, ivs, ovs, nvs, irows, orows, nrows, sp, sn, sem):
        wid = lax.axis_index("s") * NC + lax.axis_index("c")

        def chunk(c, _):
            base = wid * b_per + c * C
            pltpu.sync_copy(ii_hbm.at[pl.ds(base, C)], iv)
            pltpu.sync_copy(oi_hbm.at[pl.ds(base * W, C * W)], ov)
            pltpu.sync_copy(ni_hbm.at[pl.ds(base * N, C * N)], nv)
            for src, dst, nv_ in ((iv, ivs, C), (ov, ovs, C * W),
                                  (nv, nvs, C * N)):
                for kk in range(nv_ // L):
                    s = pl.ds(kk * L, L)
                    v = src[s]
                    dst[s] = v - _group_of(v) * QP
            cp_i = pltpu.async_copy(pin_hbm.at[ivs], irows, sem)
            cp_o = pltpu.async_copy(pout_hbm.at[ovs], orows, sem)
            cp_n = pltpu.async_copy(pout_hbm.at[nvs], nrows, sem)
            cp_i.wait()
            cp_o.wait()
            cp_n.wait()

            def group(g, _):
                bvec = g * L + lax.iota(jnp.int32, L)
                icol = _group_of(plsc.load_gather(iv, [bvec])) * E
                ocols = []
                for w in range(W):
                    ocols.append(
                        _group_of(plsc.load_gather(ov, [bvec * W + w])) * E)
                ncols = []
                for n in range(N):
                    ncols.append(
                        _group_of(plsc.load_gather(nv, [bvec * N + n])) * E)
                pacc = jnp.zeros((L,), jnp.float32)
                nacc = jnp.zeros((L,), jnp.float32)
                lane = lax.iota(jnp.int32, L)
                for e in range(E):
                    ev = jnp.bitwise_and(e + lane, E - 1)
                    v_in = plsc.load_gather(irows, [bvec, icol + ev])
                    pe = plsc.load_gather(orows, [bvec * W, ocols[0] + ev])
                    for w in range(1, W):
                        pe = pe + plsc.load_gather(
                            orows, [bvec * W + w, ocols[w] + ev])
                    ne = plsc.load_gather(nrows, [bvec * N, ncols[0] + ev])
                    for n in range(1, N):
                        ne = ne + plsc.load_gather(
                            nrows, [bvec * N + n, ncols[n] + ev])
                    pacc = pacc + v_in * pe
                    nacc = nacc + v_in * ne
                sp[pl.ds(g * L, L)] = pacc
                sn[pl.ds(g * L, L)] = nacc
                return 0

            lax.fori_loop(0, G, group, 0)
            pltpu.sync_copy(sp, spos_hbm.at[pl.ds(base, C)])
            pltpu.sync_copy(sn, sneg_hbm.at[pl.ds(base, C)])
            return 0

        lax.fori_loop(0, n_ch, chunk, 0)

    return k(pin, pout, i_idx, o_idx, n_idx)


def _tc_loss(s_pos, s_neg, B):
    def body(sp_ref, sn_ref, o_ref):
        def logsig(x):
            return jnp.minimum(x, 0.0) - jnp.log1p(jnp.exp(-jnp.abs(x)))

        o_ref[0, 0] = jnp.sum(logsig(sn_ref[...]) - logsig(sp_ref[...])) \
            * (1.0 / B)

    return pl.pallas_call(
        body,
        in_specs=[
            pl.BlockSpec(memory_space=pltpu.VMEM),
            pl.BlockSpec(memory_space=pltpu.VMEM),
        ],
        out_specs=pl.BlockSpec(memory_space=pltpu.SMEM),
        out_shape=jax.ShapeDtypeStruct((1, 1), jnp.float32),
    )(s_pos, s_neg)


def kernel(i, o, neg, in_table, out_table):
    B = i.shape[0]
    W = o.shape[1]
    N = neg.shape[1]
    E = in_table.shape[1]
    pin = _tc_repack(in_table.T, E)
    pout = _tc_repack(out_table.T, E)
    i32 = i.astype(jnp.int32)
    o32 = o.astype(jnp.int32).reshape(-1)
    n32 = neg.astype(jnp.int32).reshape(-1)
    s_pos, s_neg = _sc_scores(pin, pout, i32, o32, n32, B, E, W, N)
    loss = _tc_loss(s_pos.reshape(128, -1), s_neg.reshape(128, -1), B)
    return loss[0, 0]


# speedup vs baseline: 2.6927x; 1.1897x over previous
import functools

import jax
import jax.numpy as jnp
from jax import lax
from jax.experimental import pallas as pl
from jax.experimental.pallas import tpu as pltpu
from jax.experimental.pallas import tpu_sc as plsc

NC = 2
NS = 16
NWORK = NC * NS
L = 16
RW = 128
K4 = 2048
NB = 123
QP = K4 * NB


def _tc_repack(tt, E):
    def body(i0, i1, i2, i3, out_ref):
        xcat = jnp.concatenate(
            [i0[...], i1[...], i2[...], i3[...]], axis=0)
        eye = (lax.broadcasted_iota(jnp.int32, (RW, RW), 0)
               == lax.broadcasted_iota(jnp.int32, (RW, RW), 1)
               ).astype(jnp.float32)
        dn = (((0,), (0,)), ((), ()))
        out_ref[...] = lax.dot_general(
            xcat, eye, dn, preferred_element_type=jnp.float32)

    last_blk = 488

    return pl.pallas_call(
        body,
        grid=(NB,),
        in_specs=[
            pl.BlockSpec(
                (E, K4),
                lambda g, j=j: (0, jnp.minimum(j * NB + g, last_blk)))
            for j in range(4)
        ],
        out_specs=pl.BlockSpec((K4, RW), lambda g: (g, 0)),
        out_shape=jax.ShapeDtypeStruct((QP, RW), jnp.float32),
    )(tt, tt, tt, tt)


def _group_of(v):
    one = jnp.int32(1)
    zero = jnp.int32(0)
    return (jnp.where(v >= QP, one, zero)
            + jnp.where(v >= 2 * QP, one, zero)
            + jnp.where(v >= 3 * QP, one, zero))


def _sc_scores(pin, pout, i_idx, o_idx, n_idx, B, E, W, N):
    b_per = B // NWORK
    C = 64
    n_ch = b_per // C
    G = C // L

    mesh = plsc.VectorSubcoreMesh(core_axis_name="c", subcore_axis_name="s")

    @functools.partial(
        pl.kernel,
        out_type=(
            jax.ShapeDtypeStruct((B,), jnp.float32),
            jax.ShapeDtypeStruct((B,), jnp.float32),
        ),
        mesh=mesh,
        compiler_params=pltpu.CompilerParams(needs_layout_passes=False),
        scratch_types=[
            pltpu.VMEM((C,), jnp.int32),
            pltpu.VMEM((C * W,), jnp.int32),
            pltpu.VMEM((C * N,), jnp.int32),
            pltpu.VMEM((C,), jnp.int32),
            pltpu.VMEM((C * W,), jnp.int32),
            pltpu.VMEM((C * N,), jnp.int32),
            pltpu.VMEM((C, RW), jnp.float32),
            pltpu.VMEM((C * W, RW), jnp.float32),
            pltpu.VMEM((C * N, RW), jnp.float32),
            pltpu.VMEM((C,), jnp.float32),
            pltpu.VMEM((C,), jnp.float32),
            pltpu.SemaphoreType.DMA,
        ],
    )
    def k(pin_hbm, pout_hbm, ii_hbm, oi_hbm, ni_hbm, spos_hbm, sneg_hbm,
          iv, ov, nv, ivs, ovs, nvs, irows, orows, nrows, sp, sn, sem):
        wid = lax.axis_index("s") * NC + lax.axis_index("c")

        def chunk(c, _):
            base = wid * b_per + c * C
            pltpu.sync_copy(ii_hbm.at[pl.ds(base, C)], iv)
            pltpu.sync_copy(oi_hbm.at[pl.ds(base * W, C * W)], ov)
            pltpu.sync_copy(ni_hbm.at[pl.ds(base * N, C * N)], nv)
            for src, dst, nv_ in ((iv, ivs, C), (ov, ovs, C * W),
                                  (nv, nvs, C * N)):
                for kk in range(nv_ // L):
                    s = pl.ds(kk * L, L)
                    v = src[s]
                    dst[s] = v - _group_of(v) * QP
            cp_i = pltpu.async_copy(pin_hbm.at[ivs], irows, sem)
            cp_o = pltpu.async_copy(pout_hbm.at[ovs], orows, sem)
            cp_n = pltpu.async_copy(pout_hbm.at[nvs], nrows, sem)
            cp_i.wait()
            cp_o.wait()
            cp_n.wait()

            def group(g, _):
                bvec = g * L + lax.iota(jnp.int32, L)
                icol = _group_of(plsc.load_gather(iv, [bvec])) * E
                ocols = []
                for w in range(W):
                    ocols.append(
                        _group_of(plsc.load_gather(ov, [bvec * W + w])) * E)
                ncols = []
                for n in range(N):
                    ncols.append(
                        _group_of(plsc.load_gather(nv, [bvec * N + n])) * E)
                pacc = jnp.zeros((L,), jnp.float32)
                nacc = jnp.zeros((L,), jnp.float32)
                lane = lax.iota(jnp.int32, L)
                for e in range(E):
                    ev = jnp.bitwise_and(e + lane, E - 1)
                    v_in = plsc.load_gather(irows, [bvec, icol + ev])
                    pe = plsc.load_gather(orows, [bvec * W, ocols[0] + ev])
                    for w in range(1, W):
                        pe = pe + plsc.load_gather(
                            orows, [bvec * W + w, ocols[w] + ev])
                    ne = plsc.load_gather(nrows, [bvec * N, ncols[0] + ev])
                    for n in range(1, N):
                        ne = ne + plsc.load_gather(
                            nrows, [bvec * N + n, ncols[n] + ev])
                    pacc = pacc + v_in * pe
                    nacc = nacc + v_in * ne
                sp[pl.ds(g * L, L)] = pacc
                sn[pl.ds(g * L, L)] = nacc
                return 0

            lax.fori_loop(0, G, group, 0)
            pltpu.sync_copy(sp, spos_hbm.at[pl.ds(base, C)])
            pltpu.sync_copy(sn, sneg_hbm.at[pl.ds(base, C)])
            return 0

        lax.fori_loop(0, n_ch, chunk, 0)

    return k(pin, pout, i_idx, o_idx, n_idx)


def _tc_loss(s_pos, s_neg, B):
    def body(sp_ref, sn_ref, o_ref):
        def logsig(x):
            return jnp.minimum(x, 0.0) - jnp.log1p(jnp.exp(-jnp.abs(x)))

        o_ref[0, 0] = jnp.sum(logsig(sn_ref[...]) - logsig(sp_ref[...])) \
            * (1.0 / B)

    return pl.pallas_call(
        body,
        in_specs=[
            pl.BlockSpec(memory_space=pltpu.VMEM),
            pl.BlockSpec(memory_space=pltpu.VMEM),
        ],
        out_specs=pl.BlockSpec(memory_space=pltpu.SMEM),
        out_shape=jax.ShapeDtypeStruct((1, 1), jnp.float32),
    )(s_pos, s_neg)


def kernel(i, o, neg, in_table, out_table):
    B = i.shape[0]
    W = o.shape[1]
    N = neg.shape[1]
    E = in_table.shape[1]
    pin = _tc_repack(in_table.T, E)
    pout = _tc_repack(out_table.T, E)
    i32 = i.astype(jnp.int32)
    o32 = o.astype(jnp.int32).reshape(-1)
    n32 = neg.astype(jnp.int32).reshape(-1)
    s_pos, s_neg = _sc_scores(pin, pout, i32, o32, n32, B, E, W, N)
    loss = _tc_loss(s_pos.reshape(128, -1), s_neg.reshape(128, -1), B)
    return loss[0, 0]


# speedup vs baseline: 3.3980x; 1.2619x over previous
import functools

import jax
import jax.numpy as jnp
from jax import lax
from jax.experimental import pallas as pl
from jax.experimental.pallas import tpu as pltpu
from jax.experimental.pallas import tpu_sc as plsc

NC = 2
NS = 16
NWORK = NC * NS
L = 16
RW = 128
K4 = 4096
NB = 62
QP = K4 * NB


def _tc_repack(tt, E):
    def body(i0, i1, i2, i3, out_ref):
        xcat = jnp.concatenate(
            [i0[...], i1[...], i2[...], i3[...]], axis=0)
        eye = (lax.broadcasted_iota(jnp.int32, (RW, RW), 0)
               == lax.broadcasted_iota(jnp.int32, (RW, RW), 1)
               ).astype(jnp.float32)
        dn = (((0,), (0,)), ((), ()))
        out_ref[...] = lax.dot_general(
            xcat, eye, dn, preferred_element_type=jnp.float32)

    last_blk = 244

    return pl.pallas_call(
        body,
        grid=(NB,),
        in_specs=[
            pl.BlockSpec(
                (E, K4),
                lambda g, j=j: (0, jnp.minimum(j * NB + g, last_blk)))
            for j in range(4)
        ],
        out_specs=pl.BlockSpec((K4, RW), lambda g: (g, 0)),
        out_shape=jax.ShapeDtypeStruct((QP, RW), jnp.float32),
    )(tt, tt, tt, tt)


def _group_of(v):
    one = jnp.int32(1)
    zero = jnp.int32(0)
    return (jnp.where(v >= QP, one, zero)
            + jnp.where(v >= 2 * QP, one, zero)
            + jnp.where(v >= 3 * QP, one, zero))


def _sc_scores(pin, pout, i_idx, o_idx, n_idx, B, E, W, N):
    b_per = B // NWORK
    C = 64
    n_ch = b_per // C
    G = C // L

    mesh = plsc.VectorSubcoreMesh(core_axis_name="c", subcore_axis_name="s")

    @functools.partial(
        pl.kernel,
        out_type=(
            jax.ShapeDtypeStruct((B,), jnp.float32),
            jax.ShapeDtypeStruct((B,), jnp.float32),
        ),
        mesh=mesh,
        compiler_params=pltpu.CompilerParams(needs_layout_passes=False),
        scratch_types=[
            pltpu.VMEM((C,), jnp.int32),
            pltpu.VMEM((C * W,), jnp.int32),
            pltpu.VMEM((C * N,), jnp.int32),
            pltpu.VMEM((C,), jnp.int32),
            pltpu.VMEM((C * W,), jnp.int32),
            pltpu.VMEM((C * N,), jnp.int32),
            pltpu.VMEM((C, RW), jnp.float32),
            pltpu.VMEM((C * W, RW), jnp.float32),
            pltpu.VMEM((C * N, RW), jnp.float32),
            pltpu.VMEM((C,), jnp.float32),
            pltpu.VMEM((C,), jnp.float32),
            pltpu.SemaphoreType.DMA,
        ],
    )
    def k(pin_hbm, pout_hbm, ii_hbm, oi_hbm, ni_hbm, spos_hbm, sneg_hbm,
          iv, ov, nv, ivs, ovs, nvs, irows, orows, nrows, sp, sn, sem):
        wid = lax.axis_index("s") * NC + lax.axis_index("c")

        def chunk(c, _):
            base = wid * b_per + c * C
            pltpu.sync_copy(ii_hbm.at[pl.ds(base, C)], iv)
            pltpu.sync_copy(oi_hbm.at[pl.ds(base * W, C * W)], ov)
            pltpu.sync_copy(ni_hbm.at[pl.ds(base * N, C * N)], nv)
            for src, dst, nv_ in ((iv, ivs, C), (ov, ovs, C * W),
                                  (nv, nvs, C * N)):
                for kk in range(nv_ // L):
                    s = pl.ds(kk * L, L)
                    v = src[s]
                    dst[s] = v - _group_of(v) * QP
            cp_i = pltpu.async_copy(pin_hbm.at[ivs], irows, sem)
            cp_o = pltpu.async_copy(pout_hbm.at[ovs], orows, sem)
            cp_n = pltpu.async_copy(pout_hbm.at[nvs], nrows, sem)
            cp_i.wait()
            cp_o.wait()
            cp_n.wait()

            def group(g, _):
                bvec = g * L + lax.iota(jnp.int32, L)
                icol = _group_of(plsc.load_gather(iv, [bvec])) * E
                ocols = []
                for w in range(W):
                    ocols.append(
                        _group_of(plsc.load_gather(ov, [bvec * W + w])) * E)
                ncols = []
                for n in range(N):
                    ncols.append(
                        _group_of(plsc.load_gather(nv, [bvec * N + n])) * E)
                pacc = jnp.zeros((L,), jnp.float32)
                nacc = jnp.zeros((L,), jnp.float32)
                lane = lax.iota(jnp.int32, L)
                for e in range(E):
                    ev = jnp.bitwise_and(e + lane, E - 1)
                    v_in = plsc.load_gather(irows, [bvec, icol + ev])
                    pe = plsc.load_gather(orows, [bvec * W, ocols[0] + ev])
                    for w in range(1, W):
                        pe = pe + plsc.load_gather(
                            orows, [bvec * W + w, ocols[w] + ev])
                    ne = plsc.load_gather(nrows, [bvec * N, ncols[0] + ev])
                    for n in range(1, N):
                        ne = ne + plsc.load_gather(
                            nrows, [bvec * N + n, ncols[n] + ev])
                    pacc = pacc + v_in * pe
                    nacc = nacc + v_in * ne
                sp[pl.ds(g * L, L)] = pacc
                sn[pl.ds(g * L, L)] = nacc
                return 0

            lax.fori_loop(0, G, group, 0)
            pltpu.sync_copy(sp, spos_hbm.at[pl.ds(base, C)])
            pltpu.sync_copy(sn, sneg_hbm.at[pl.ds(base, C)])
            return 0

        lax.fori_loop(0, n_ch, chunk, 0)

    return k(pin, pout, i_idx, o_idx, n_idx)


def _tc_loss(s_pos, s_neg, B):
    def body(sp_ref, sn_ref, o_ref):
        def logsig(x):
            return jnp.minimum(x, 0.0) - jnp.log1p(jnp.exp(-jnp.abs(x)))

        o_ref[0, 0] = jnp.sum(logsig(sn_ref[...]) - logsig(sp_ref[...])) \
            * (1.0 / B)

    return pl.pallas_call(
        body,
        in_specs=[
            pl.BlockSpec(memory_space=pltpu.VMEM),
            pl.BlockSpec(memory_space=pltpu.VMEM),
        ],
        out_specs=pl.BlockSpec(memory_space=pltpu.SMEM),
        out_shape=jax.ShapeDtypeStruct((1, 1), jnp.float32),
    )(s_pos, s_neg)


def kernel(i, o, neg, in_table, out_table):
    B = i.shape[0]
    W = o.shape[1]
    N = neg.shape[1]
    E = in_table.shape[1]
    pin = _tc_repack(in_table.T, E)
    pout = _tc_repack(out_table.T, E)
    i32 = i.astype(jnp.int32)
    o32 = o.astype(jnp.int32).reshape(-1)
    n32 = neg.astype(jnp.int32).reshape(-1)
    s_pos, s_neg = _sc_scores(pin, pout, i32, o32, n32, B, E, W, N)
    loss = _tc_loss(s_pos.reshape(128, -1), s_neg.reshape(128, -1), B)
    return loss[0, 0]
